# speedup vs baseline: 5.3013x; 5.3013x over previous
import jax
import jax.numpy as jnp
from jax.experimental import pallas as pl
from jax.experimental.pallas import tpu as pltpu

_B, _T, _D, _C = 10, 2048, 2048, 20
_K = 292
_DT = 512
_ND = _D // _DT
_NPAIR = 3
_INT_MIN = -(2 ** 31)
_M31 = 2 ** 31 - 1


def _sortable(bits):
    return jnp.where(bits < 0, bits ^ _M31, bits)


def _topk_mean(x):
    keys = _sortable(jax.lax.bitcast_convert_type(x, jnp.int32))
    kf = jnp.float32(_K)

    def cnt_ge(c):
        return jnp.sum((keys >= c).astype(jnp.float32), axis=1, keepdims=True)

    rows = x.shape[0]
    zero = jnp.zeros((rows, 1), jnp.int32)
    t0 = jnp.where(cnt_ge(zero) >= kf, zero,
                   jnp.full((rows, 1), _INT_MIN, jnp.int32))

    def body(b, t):
        bit = jax.lax.shift_left(jnp.int32(1), jnp.int32(30) - b)
        cand = t + bit
        return jnp.where(cnt_ge(cand) >= kf, cand, t)

    t = jax.lax.fori_loop(jnp.int32(0), jnp.int32(31), body, t0)
    gt = keys > t
    cnt_gt = jnp.sum(gt.astype(jnp.float32), axis=1, keepdims=True)
    ssum = jnp.sum(jnp.where(gt, x, jnp.float32(0.0)), axis=1, keepdims=True)
    tval = jax.lax.bitcast_convert_type(_sortable(t), jnp.float32)
    return (ssum + (kf - cnt_gt) * tval) * jnp.float32(1.0 / _K)


def _mil(il, lab):
    lwb = lab * (1.0 / (jnp.sum(lab) + 1e-4))
    mx = jnp.max(il)
    ls = il - mx - jnp.log(jnp.sum(jnp.exp(il - mx)))
    return -jnp.sum(lwb * ls)


def _stats_kernel(cas_ref, atn_ref, mask_ref, v_ref, f_ref, labb_ref,
                  labs_ref, scal_ref, a_ref):
    mil_orig = jnp.float32(0.0)
    mil_supp = jnp.float32(0.0)
    mutual = jnp.float32(0.0)
    norm_a = jnp.float32(0.0)
    norm_v = jnp.float32(0.0)
    norm_f = jnp.float32(0.0)
    guide_a = jnp.float32(0.0)
    guide_v = jnp.float32(0.0)
    guide_f = jnp.float32(0.0)
    inv_t = jnp.float32(1.0 / _T)
    for i in range(_B):
        cas = cas_ref[i]
        atn = atn_ref[i]
        msk = mask_ref[i]
        v = v_ref[i] * msk
        f = f_ref[i] * msk
        el = cas * msk
        atn_m = atn * msk
        mutual += jnp.mean((v - f) ** 2)
        mn = jnp.min(el, axis=0, keepdims=True)
        supp = atn_m * (el - mn) + mn
        mil_orig += _mil(_topk_mean(el), labb_ref[i])
        mil_supp += _mil(_topk_mean(supp), labs_ref[i])
        mx = jnp.max(el, axis=0, keepdims=True)
        z = jnp.sum(jnp.exp(el - mx), axis=0, keepdims=True)
        bg = jnp.exp(el[_C:_C + 1, :] - mx) / z
        norm_a += jnp.sum(atn_m) * inv_t
        norm_v += jnp.sum(v) * inv_t
        norm_f += jnp.sum(f) * inv_t
        guide_a += jnp.sum(jnp.abs(1.0 - atn_m - bg)) * inv_t
        guide_v += jnp.sum(jnp.abs(1.0 - v - bg)) * inv_t
        guide_f += jnp.sum(jnp.abs(1.0 - f - bg)) * inv_t
        if i < 2 * _NPAIR:
            mnu = jnp.min(cas, axis=0, keepdims=True)
            st = atn * (cas - mnu) + mnu
            mxt = jnp.max(st, axis=1, keepdims=True)
            e = jnp.exp(st - mxt)
            zt = jnp.sum(e, axis=1, keepdims=True)
            a1 = e / zt
            al = (1.0 - a1) * jnp.float32(1.0 / (_T - 1))
            pad = jnp.zeros((3, _T), jnp.float32)
            a_ref[i] = jnp.concatenate([a1, pad, al, pad], axis=0)
    packed = jnp.concatenate(
        [jnp.reshape(s, (1, 1)) for s in
         (mil_orig, mil_supp, mutual, norm_a, norm_v, norm_f,
          guide_a, guide_v, guide_f)] + [jnp.zeros((1, 7), jnp.float32)],
        axis=1)
    scal_ref[...] = packed


def _contrast_kernel(a1_ref, a2_ref, x1_ref, x2_ref, l1_ref, l2_ref,
                     out_ref, acc_ref):
    d = pl.program_id(1)

    @pl.when(d == 0)
    def _zero():
        acc_ref[...] = jnp.zeros_like(acc_ref)

    dn = (((1,), (0,)), ((), ()))
    m1 = jax.lax.dot_general(a1_ref[0], x1_ref[0], dn,
                             preferred_element_type=jnp.float32,
                             precision=jax.lax.Precision.HIGHEST)
    m2 = jax.lax.dot_general(a2_ref[0], x2_ref[0], dn,
                             preferred_element_type=jnp.float32,
                             precision=jax.lax.Precision.HIGHEST)
    h1 = m1[0:_C + 1]
    l1 = m1[24:24 + _C + 1]
    h2 = m2[0:_C + 1]
    l2 = m2[24:24 + _C + 1]
    acc_ref[0] = acc_ref[0] + h1 * h2
    acc_ref[1] = acc_ref[1] + h1 * l2
    acc_ref[2] = acc_ref[2] + h2 * l1
    acc_ref[3] = acc_ref[3] + h1 * h1
    acc_ref[4] = acc_ref[4] + h2 * h2
    acc_ref[5] = acc_ref[5] + l1 * l1
    acc_ref[6] = acc_ref[6] + l2 * l2

    @pl.when(d == _ND - 1)
    def _final():
        h1h2 = jnp.sum(acc_ref[0], axis=1, keepdims=True)
        h1l2 = jnp.sum(acc_ref[1], axis=1, keepdims=True)
        h2l1 = jnp.sum(acc_ref[2], axis=1, keepdims=True)
        nh1 = jnp.sqrt(jnp.sum(acc_ref[3], axis=1, keepdims=True))
        nh2 = jnp.sqrt(jnp.sum(acc_ref[4], axis=1, keepdims=True))
        nl1 = jnp.sqrt(jnp.sum(acc_ref[5], axis=1, keepdims=True))
        nl2 = jnp.sqrt(jnp.sum(acc_ref[6], axis=1, keepdims=True))
        d1 = 1.0 - h1h2 / (nh1 * nh2)
        d2 = 1.0 - h1l2 / (nh1 * nl2)
        d3 = 1.0 - h2l1 / (nh2 * nl1)
        ll = l1_ref[0] * l2_ref[0]
        part = 0.5 * (jnp.sum(jnp.maximum(d1 - d2 + 0.5, 0.0) * ll)
                      + jnp.sum(jnp.maximum(d1 - d3 + 0.5, 0.0) * ll))
        ntmp = jnp.sum(ll)
        out_ref[0] = jnp.concatenate(
            [jnp.reshape(part, (1, 1)), jnp.reshape(ntmp, (1, 1))], axis=1)


def kernel(feat, cas, attn, mask, v_atn, f_atn, labels):
    f32 = jnp.float32
    cas_t = jnp.transpose(cas, (0, 2, 1))
    atn_t = jnp.transpose(attn, (0, 2, 1))
    mask_t = jnp.transpose(mask, (0, 2, 1))
    v_t = jnp.transpose(v_atn, (0, 2, 1))
    f_t = jnp.transpose(f_atn, (0, 2, 1))
    labb = jnp.concatenate([labels, jnp.ones_like(labels[:, :1])], axis=1)[:, :, None]
    labs = jnp.concatenate([labels, jnp.zeros_like(labels[:, :1])], axis=1)[:, :, None]

    scal, amats = pl.pallas_call(
        _stats_kernel,
        out_shape=(jax.ShapeDtypeStruct((1, 16), f32),
                   jax.ShapeDtypeStruct((2 * _NPAIR, 48, _T), f32)),
    )(cas_t, atn_t, mask_t, v_t, f_t, labb, labs)

    pairs = pl.pallas_call(
        _contrast_kernel,
        grid=(_NPAIR, _ND),
        in_specs=[
            pl.BlockSpec((1, 48, _T), lambda p, d: (2 * p, 0, 0)),
            pl.BlockSpec((1, 48, _T), lambda p, d: (2 * p + 1, 0, 0)),
            pl.BlockSpec((1, _T, _DT), lambda p, d: (2 * p, 0, d)),
            pl.BlockSpec((1, _T, _DT), lambda p, d: (2 * p + 1, 0, d)),
            pl.BlockSpec((1, _C + 1, 1), lambda p, d: (2 * p, 0, 0)),
            pl.BlockSpec((1, _C + 1, 1), lambda p, d: (2 * p + 1, 0, 0)),
        ],
        out_specs=pl.BlockSpec((1, 1, 2), lambda p, d: (p, 0, 0)),
        out_shape=jax.ShapeDtypeStruct((_NPAIR, 1, 2), f32),
        scratch_shapes=[pltpu.VMEM((7, _C + 1, _DT), f32)],
    )(amats, amats, feat, feat, labs, labs)

    loss_contrastive = jnp.sum(pairs[:, 0, 0]) / jnp.sum(pairs[:, 0, 1])
    s = scal[0]
    inv = f32(0.1)
    mil_orig = s[0] * inv
    mil_supp = s[1] * inv
    mutual = s[2] * inv
    norm_avg = (s[3] + s[4] + s[5]) * (inv / 3.0)
    guide_avg = (s[6] + s[7] + s[8]) * (inv / 3.0)
    total = (mil_orig + mil_supp + loss_contrastive + mutual
             + 0.8 * norm_avg + 0.8 * guide_avg)
    return (total, mil_orig, mil_supp, loss_contrastive, mutual,
            norm_avg, guide_avg)
